# Initial kernel scaffold; baseline (speedup 1.0000x reference)
import jax
import jax.numpy as jnp
from jax.experimental import pallas as pl


def kernel(feat, edge_index, distances, interaction_cutoffs, rbf_kernel_means, rbf_kernel_scaling, features_to_use):
    raise NotImplementedError("write your pallas kernel here")


# speedup vs baseline: 442.2952x; 442.2952x over previous
import jax
import jax.numpy as jnp
import numpy as np
from jax import lax
from jax.experimental import pallas as pl
from jax.experimental.pallas import tpu as pltpu
from jax.experimental.pallas import tpu_sc as plsc

N = 100000
E = 3200000
K = 3
T = 4

NC = 2
NS = 16
NW = NC * NS
L = 16

EPW = E // NW
CHUNK = 2000
GRP = 80
NSUB = CHUNK // GRP
NG16 = GRP // L
NCHUNKS = EPW // CHUNK

NPAD = 131072
NQ = NPAD // 8
W = 16
RPT = 6256
RPT_LAST = 6160

_F32 = jnp.float32
_I32 = jnp.int32


def _radial_body(d_ref, cut_ref, mean_ref, scal_ref, o_ref):
    d = d_ref[...]
    pi = np.float32(np.pi)
    for k in range(K):
        c = cut_ref[k]
        m = mean_ref[k]
        s = scal_ref[k]
        rbf = jnp.exp(-s * (d - m) ** 2)
        cosv = 0.5 * (jnp.cos(pi * d / c) + 1.0)
        w = jnp.where(d <= c, cosv, jnp.zeros_like(cosv))
        o_ref[k] = rbf * w


def _radial(d2, cut, mean, scal):
    nrows = d2.shape[0]
    grid = nrows // 8
    return pl.pallas_call(
        _radial_body,
        out_shape=jax.ShapeDtypeStruct((K, nrows, d2.shape[1]), _F32),
        grid=(grid,),
        in_specs=[
            pl.BlockSpec((8, d2.shape[1]), lambda i: (i, 0)),
            pl.BlockSpec(memory_space=pltpu.SMEM),
            pl.BlockSpec(memory_space=pltpu.SMEM),
            pl.BlockSpec(memory_space=pltpu.SMEM),
        ],
        out_specs=pl.BlockSpec((K, 8, d2.shape[1]), lambda i: (0, i, 0)),
    )(d2, cut, mean, scal)


def _colmap_body(f_ref, ftu_ref, o_ref):
    w = jnp.zeros((8, 128), _I32)
    for j in range(8):
        f = f_ref[j]
        slot = jnp.full(f.shape, 15, _I32)
        for t in range(T - 1, -1, -1):
            slot = jnp.where(f == ftu_ref[t], jnp.full_like(slot, t), slot)
        w = w | (slot << (4 * j))
    o_ref[...] = w


def _colmap(fpad3, ftu):
    grid = fpad3.shape[1] // 8
    return pl.pallas_call(
        _colmap_body,
        out_shape=jax.ShapeDtypeStruct((fpad3.shape[1], 128), _I32),
        grid=(grid,),
        in_specs=[
            pl.BlockSpec((8, 8, 128), lambda i: (0, i, 0)),
            pl.BlockSpec(memory_space=pltpu.SMEM),
        ],
        out_specs=pl.BlockSpec((8, 128), lambda i: (i, 0)),
    )(fpad3, ftu)


def _sc_body(colq_hbm, src_hbm, dst_hbm, he_hbm, out_hbm,
             colq_v, src_v, dst_v, he_v, dst80_v, contrib_v, stg_v, acc_sh):
    cid = lax.axis_index("c")
    sid = lax.axis_index("s")
    wid = cid * NS + sid

    for i in range(128):
        stg_v[i, :] = jnp.zeros((L,), _F32)
    r0 = pl.multiple_of(sid * RPT, 16)

    def _zero(i, carry):
        pltpu.sync_copy(stg_v, acc_sh.at[pl.ds(r0 + i * 128, 128)])
        return carry

    lax.fori_loop(0, 48, _zero, 0)
    pltpu.sync_copy(stg_v.at[pl.ds(0, 16)], acc_sh.at[pl.ds(r0 + 6144, 16)])

    @pl.when(sid < NS - 1)
    def _zero_tail():
        pltpu.sync_copy(stg_v.at[pl.ds(0, 96)],
                        acc_sh.at[pl.ds(r0 + RPT_LAST, 96)])

    pltpu.sync_copy(colq_hbm, colq_v)
    plsc.subcore_barrier()

    e0 = wid * EPW
    iota = lax.iota(_I32, L)

    def _chunk(ch, carry):
        off = pl.multiple_of(e0 + ch * CHUNK, 8)
        pltpu.sync_copy(src_hbm.at[pl.ds(off, CHUNK)], src_v)
        pltpu.sync_copy(dst_hbm.at[pl.ds(off, CHUNK)], dst_v)
        pltpu.sync_copy(he_hbm.at[pl.ds(pl.multiple_of(3 * off, 8), 3 * CHUNK)],
                        he_v)

        def _sub(j, carry2):
            b0 = j * GRP
            for g in range(NG16):
                b = b0 + g * L
                ev = iota + b
                srcv = src_v[pl.ds(b, L)]
                wq = plsc.load_gather(colq_v, [srcv & 16383])
                shift = (srcv >> 12) & 28
                colv = ((wq >> shift) & 15) * 3
                he0 = plsc.load_gather(he_v, [3 * ev])
                he1 = plsc.load_gather(he_v, [3 * ev + 1])
                he2 = plsc.load_gather(he_v, [3 * ev + 2])
                hek = (he0, he1, he2)
                row = iota + g * L
                for t in range(T):
                    msk = colv == 3 * t
                    for k in range(K):
                        v = jnp.where(msk, hek[k], jnp.zeros((L,), _F32))
                        colc = jnp.full((L,), 3 * t + k, _I32)
                        plsc.store_scatter(contrib_v, [row, colc], v)
                dst80_v[pl.ds(g * L, L)] = dst_v[pl.ds(b, L)]
            pltpu.sync_copy(contrib_v, acc_sh.at[dst80_v], add=True)
            return carry2

        lax.fori_loop(0, NSUB, _sub, 0)
        return carry

    lax.fori_loop(0, NCHUNKS, _chunk, 0)

    plsc.subcore_barrier()

    o0 = pl.multiple_of(cid * N + r0, 16)

    def _wb(i, carry):
        pltpu.sync_copy(acc_sh.at[pl.ds(r0 + i * 128, 128)], stg_v)
        pltpu.sync_copy(stg_v, out_hbm.at[pl.ds(o0 + i * 128, 128)])
        return carry

    lax.fori_loop(0, 48, _wb, 0)
    pltpu.sync_copy(acc_sh.at[pl.ds(r0 + 6144, 16)], stg_v.at[pl.ds(0, 16)])
    pltpu.sync_copy(stg_v.at[pl.ds(0, 16)], out_hbm.at[pl.ds(o0 + 6144, 16)])

    @pl.when(sid < NS - 1)
    def _wb_tail():
        pltpu.sync_copy(acc_sh.at[pl.ds(r0 + RPT_LAST, 96)],
                        stg_v.at[pl.ds(0, 96)])
        pltpu.sync_copy(stg_v.at[pl.ds(0, 96)],
                        out_hbm.at[pl.ds(o0 + RPT_LAST, 96)])


def _sc_call(colq, src, dst, he_flat):
    mesh = plsc.VectorSubcoreMesh(core_axis_name="c", subcore_axis_name="s")
    k = pl.kernel(
        _sc_body,
        out_type=jax.ShapeDtypeStruct((NC * N, W), _F32),
        mesh=mesh,
        scratch_types=[
            pltpu.VMEM((NQ,), _I32),
            pltpu.VMEM((CHUNK,), _I32),
            pltpu.VMEM((CHUNK,), _I32),
            pltpu.VMEM((3 * CHUNK,), _F32),
            pltpu.VMEM((GRP,), _I32),
            pltpu.VMEM((GRP, W), _F32),
            pltpu.VMEM((128, W), _F32),
            pltpu.VMEM_SHARED((N, W), _F32),
        ],
        compiler_params=pltpu.CompilerParams(needs_layout_passes=False,
                                             use_tc_tiling_on_sc=False),
    )
    return k(colq, src, dst, he_flat)


def _reduce_body(p_ref, o_ref):
    o_ref[...] = p_ref[0] + p_ref[1]


def _reduce(parts3):
    nrows = parts3.shape[1]
    return pl.pallas_call(
        _reduce_body,
        out_shape=jax.ShapeDtypeStruct((nrows, 128), _F32),
    )(parts3)


def kernel(feat, edge_index, distances, interaction_cutoffs, rbf_kernel_means,
           rbf_kernel_scaling, features_to_use):
    d2 = distances.reshape(1000, 3200)
    cut = interaction_cutoffs.reshape(K)
    mean = rbf_kernel_means.reshape(K)
    scal = rbf_kernel_scaling.reshape(K)
    radial = _radial(d2, cut, mean, scal)
    he_flat = radial.reshape(3 * E)

    fpad = jnp.pad(feat.reshape(N), (0, NPAD - N)).reshape(8, NQ // 128, 128)
    colq = _colmap(fpad, features_to_use).reshape(NQ)

    src = edge_index[0].astype(_I32)
    dst = edge_index[1].astype(_I32)

    parts = _sc_call(colq, src, dst, he_flat)
    s = _reduce(parts.reshape(2, (N * W) // 128, 128))
    return s.reshape(N, W)[:, :12]


# speedup vs baseline: 614.3826x; 1.3891x over previous
import jax
import jax.numpy as jnp
import numpy as np
from jax import lax
from jax.experimental import pallas as pl
from jax.experimental.pallas import tpu as pltpu
from jax.experimental.pallas import tpu_sc as plsc

N = 100000
E = 3200000
K = 3
T = 4

NC = 2
NS = 16
NW = NC * NS
L = 16

EPW = E // NW
CHUNK = 800
GRP = 80
NSUB = CHUNK // GRP
NG16 = GRP // L
NCHUNKS = EPW // CHUNK

NPAD = 131072
NQ = NPAD // 8
W = 16
RPT = 6256
RPT_LAST = 6160

_F32 = jnp.float32
_I32 = jnp.int32


def _radial_body(d_ref, cut_ref, mean_ref, scal_ref, o_ref):
    d = d_ref[...]
    pi = np.float32(np.pi)
    for k in range(K):
        c = cut_ref[k]
        m = mean_ref[k]
        s = scal_ref[k]
        rbf = jnp.exp(-s * (d - m) ** 2)
        cosv = 0.5 * (jnp.cos(pi * d / c) + 1.0)
        w = jnp.where(d <= c, cosv, jnp.zeros_like(cosv))
        o_ref[k] = rbf * w


def _radial(d2, cut, mean, scal):
    nrows = d2.shape[0]
    grid = nrows // 8
    return pl.pallas_call(
        _radial_body,
        out_shape=jax.ShapeDtypeStruct((K, nrows, d2.shape[1]), _F32),
        grid=(grid,),
        in_specs=[
            pl.BlockSpec((8, d2.shape[1]), lambda i: (i, 0)),
            pl.BlockSpec(memory_space=pltpu.SMEM),
            pl.BlockSpec(memory_space=pltpu.SMEM),
            pl.BlockSpec(memory_space=pltpu.SMEM),
        ],
        out_specs=pl.BlockSpec((K, 8, d2.shape[1]), lambda i: (0, i, 0)),
    )(d2, cut, mean, scal)


def _colmap_body(f_ref, ftu_ref, o_ref):
    w = jnp.zeros((8, 128), _I32)
    for j in range(8):
        f = f_ref[j]
        slot = jnp.full(f.shape, 15, _I32)
        for t in range(T - 1, -1, -1):
            slot = jnp.where(f == ftu_ref[t], jnp.full_like(slot, t), slot)
        w = w | (slot << (4 * j))
    o_ref[...] = w


def _colmap(fpad3, ftu):
    grid = fpad3.shape[1] // 8
    return pl.pallas_call(
        _colmap_body,
        out_shape=jax.ShapeDtypeStruct((fpad3.shape[1], 128), _I32),
        grid=(grid,),
        in_specs=[
            pl.BlockSpec((8, 8, 128), lambda i: (0, i, 0)),
            pl.BlockSpec(memory_space=pltpu.SMEM),
        ],
        out_specs=pl.BlockSpec((8, 128), lambda i: (i, 0)),
    )(fpad3, ftu)


def _sc_body(colq_hbm, src_hbm, dst_hbm, he_hbm, out_hbm,
             colq_v, src_v, dst_v, he_v, dst80_v, contrib_v, acc_sh,
             sem_ld, sem_sc):
    cid = lax.axis_index("c")
    sid = lax.axis_index("s")
    wid = cid * NS + sid

    for p in range(2):
        for i in range(GRP):
            contrib_v[p, i, :] = jnp.zeros((L,), _F32)
        for g in range(NG16):
            dst80_v[p, pl.ds(g * L, L)] = jnp.zeros((L,), _I32)

    r0 = pl.multiple_of(sid * RPT, 16)
    z0 = contrib_v.at[0]

    def _zero(i, carry):
        pltpu.sync_copy(z0, acc_sh.at[pl.ds(r0 + i * GRP, GRP)])
        return carry

    lax.fori_loop(0, 77, _zero, 0)

    @pl.when(sid < NS - 1)
    def _zero_tail():
        pltpu.sync_copy(z0, acc_sh.at[pl.ds(r0 + 6160, 80)])
        pltpu.sync_copy(z0.at[pl.ds(0, 16)], acc_sh.at[pl.ds(r0 + 6240, 16)])

    pltpu.sync_copy(colq_hbm, colq_v)
    plsc.subcore_barrier()

    e0 = wid * EPW
    iota = lax.iota(_I32, L)

    def _issue_loads(ch, p):
        off = pl.multiple_of(e0 + ch * CHUNK, 8)
        pltpu.async_copy(src_hbm.at[pl.ds(off, CHUNK)], src_v.at[p],
                         sem_ld.at[p])
        pltpu.async_copy(dst_hbm.at[pl.ds(off, CHUNK)], dst_v.at[p],
                         sem_ld.at[p])
        pltpu.async_copy(
            he_hbm.at[pl.ds(pl.multiple_of(3 * (e0 + ch * CHUNK), 8),
                            3 * CHUNK)],
            he_v.at[p], sem_ld.at[p])

    def _wait_loads(ch, p):
        off = pl.multiple_of(e0 + ch * CHUNK, 8)
        pltpu.make_async_copy(src_hbm.at[pl.ds(off, CHUNK)], src_v.at[p],
                              sem_ld.at[p]).wait()
        pltpu.make_async_copy(dst_hbm.at[pl.ds(off, CHUNK)], dst_v.at[p],
                              sem_ld.at[p]).wait()
        pltpu.make_async_copy(
            he_hbm.at[pl.ds(pl.multiple_of(3 * (e0 + ch * CHUNK), 8),
                            3 * CHUNK)],
            he_v.at[p], sem_ld.at[p]).wait()

    def _issue_scatter(p):
        pltpu.async_copy(contrib_v.at[p], acc_sh.at[dst80_v.at[p]],
                         sem_sc.at[p], add=True)

    def _wait_scatter(p):
        pltpu.make_async_copy(contrib_v.at[p], acc_sh.at[dst80_v.at[p]],
                              sem_sc.at[p]).wait()

    _issue_scatter(0)
    _issue_scatter(1)
    _issue_loads(0, 0)

    def _chunk(ch, carry):
        cp = ch & 1
        _wait_loads(ch, cp)

        @pl.when(ch < NCHUNKS - 1)
        def _prefetch():
            _issue_loads(ch + 1, 1 - cp)

        srcc = src_v.at[cp]
        dstc = dst_v.at[cp]
        hec = he_v.at[cp]

        def _sub(j, carry2):
            p = (ch * NSUB + j) & 1
            _wait_scatter(p)
            b0 = j * GRP
            for g in range(NG16):
                b = b0 + g * L
                ev = iota + b
                srcv = srcc[pl.ds(b, L)]
                wq = plsc.load_gather(colq_v, [srcv & 16383])
                shift = (srcv >> 12) & 28
                colv = ((wq >> shift) & 15) * 3
                he0 = plsc.load_gather(hec, [3 * ev])
                he1 = plsc.load_gather(hec, [3 * ev + 1])
                he2 = plsc.load_gather(hec, [3 * ev + 2])
                hek = (he0, he1, he2)
                row = iota + g * L
                for t in range(T):
                    msk = colv == 3 * t
                    for k in range(K):
                        v = jnp.where(msk, hek[k], jnp.zeros((L,), _F32))
                        colc = jnp.full((L,), 3 * t + k, _I32)
                        plsc.store_scatter(contrib_v.at[p], [row, colc], v)
                dst80_v[p, pl.ds(g * L, L)] = dstc[pl.ds(b, L)]
            _issue_scatter(p)
            return carry2

        lax.fori_loop(0, NSUB, _sub, 0)
        return carry

    lax.fori_loop(0, NCHUNKS, _chunk, 0)

    _wait_scatter(0)
    _wait_scatter(1)
    plsc.subcore_barrier()

    o0 = pl.multiple_of(cid * N + r0, 16)
    stg = contrib_v.at[0]

    def _wb(i, carry):
        pltpu.sync_copy(acc_sh.at[pl.ds(r0 + i * GRP, GRP)], stg)
        pltpu.sync_copy(stg, out_hbm.at[pl.ds(o0 + i * GRP, GRP)])
        return carry

    lax.fori_loop(0, 77, _wb, 0)

    @pl.when(sid < NS - 1)
    def _wb_tail():
        pltpu.sync_copy(acc_sh.at[pl.ds(r0 + 6160, 80)], stg)
        pltpu.sync_copy(stg, out_hbm.at[pl.ds(o0 + 6160, 80)])
        pltpu.sync_copy(acc_sh.at[pl.ds(r0 + 6240, 16)], stg.at[pl.ds(0, 16)])
        pltpu.sync_copy(stg.at[pl.ds(0, 16)], out_hbm.at[pl.ds(o0 + 6240, 16)])


def _sc_call(colq, src, dst, he_flat):
    mesh = plsc.VectorSubcoreMesh(core_axis_name="c", subcore_axis_name="s")
    k = pl.kernel(
        _sc_body,
        out_type=jax.ShapeDtypeStruct((NC * N, W), _F32),
        mesh=mesh,
        scratch_types=[
            pltpu.VMEM((NQ,), _I32),
            pltpu.VMEM((2, CHUNK), _I32),
            pltpu.VMEM((2, CHUNK), _I32),
            pltpu.VMEM((2, 3 * CHUNK), _F32),
            pltpu.VMEM((2, GRP), _I32),
            pltpu.VMEM((2, GRP, W), _F32),
            pltpu.VMEM_SHARED((N, W), _F32),
            pltpu.SemaphoreType.DMA((2,)),
            pltpu.SemaphoreType.DMA((2,)),
        ],
        compiler_params=pltpu.CompilerParams(needs_layout_passes=False,
                                             use_tc_tiling_on_sc=False),
    )
    return k(colq, src, dst, he_flat)


def _reduce_body(p_ref, o_ref):
    o_ref[...] = p_ref[0] + p_ref[1]


def _reduce(parts3):
    nrows = parts3.shape[1]
    return pl.pallas_call(
        _reduce_body,
        out_shape=jax.ShapeDtypeStruct((nrows, 128), _F32),
    )(parts3)


def kernel(feat, edge_index, distances, interaction_cutoffs, rbf_kernel_means,
           rbf_kernel_scaling, features_to_use):
    d2 = distances.reshape(1000, 3200)
    cut = interaction_cutoffs.reshape(K)
    mean = rbf_kernel_means.reshape(K)
    scal = rbf_kernel_scaling.reshape(K)
    radial = _radial(d2, cut, mean, scal)
    he_flat = radial.reshape(3 * E)

    fpad = jnp.pad(feat.reshape(N), (0, NPAD - N)).reshape(8, NQ // 128, 128)
    colq = _colmap(fpad, features_to_use).reshape(NQ)

    src = edge_index[0].astype(_I32)
    dst = edge_index[1].astype(_I32)

    parts = _sc_call(colq, src, dst, he_flat)
    s = _reduce(parts.reshape(2, (N * W) // 128, 128))
    return s.reshape(N, W)[:, :12]
